# baseline (speedup 1.0000x reference)
import jax
import jax.numpy as jnp
from jax.experimental import pallas as pl
from jax.experimental.pallas import tpu as pltpu
from jax.experimental.pallas import tpu_sc as plsc

LANES = 128


def kernel(input_ids, weight):
    B, T = input_ids.shape
    D = weight.shape[1]
    ids = input_ids.astype(jnp.int32)
    wp = jnp.tile(weight, (1, LANES // D))

    mesh = plsc.VectorSubcoreMesh(core_axis_name="core", subcore_axis_name="subcore")

    @pl.kernel(
        out_type=jax.ShapeDtypeStruct((B, T, LANES), weight.dtype),
        mesh=mesh,
    )
    def gather_kernel(w_hbm, i_hbm, o_hbm):
        def body(i_vmem, o_vmem):
            pltpu.sync_copy(w_hbm.at[i_vmem.at[0]], o_vmem.at[0])

        pltpu.emit_pipeline(
            body,
            grid=(B,),
            in_specs=[pl.BlockSpec((1, T), index_map=lambda i: (i, 0))],
            out_specs=[pl.BlockSpec((1, T, LANES), index_map=lambda i: (i, 0, 0))],
            core_axis_name=("core", "subcore"),
            dimension_semantics=(pltpu.PARALLEL,),
        )(i_hbm, o_hbm)

    return gather_kernel(wp, ids)[:, :, :D]


# speedup vs baseline: 1.1442x; 1.1442x over previous
import jax
import jax.numpy as jnp
from jax.experimental import pallas as pl
from jax.experimental.pallas import tpu as pltpu
from jax.experimental.pallas import tpu_sc as plsc

LANES = 128


def kernel(input_ids, weight):
    B, T = input_ids.shape
    D = weight.shape[1]
    ids = input_ids.astype(jnp.int32)
    wp = jnp.pad(weight.T, ((0, LANES - D), (0, 0))).T

    mesh = plsc.VectorSubcoreMesh(core_axis_name="core", subcore_axis_name="subcore")

    @pl.kernel(
        out_type=jax.ShapeDtypeStruct((B, T, LANES), weight.dtype),
        mesh=mesh,
    )
    def gather_kernel(w_hbm, i_hbm, o_hbm):
        def body(i_vmem, o_vmem):
            pltpu.sync_copy(w_hbm.at[i_vmem.at[0]], o_vmem.at[0])

        pltpu.emit_pipeline(
            body,
            grid=(B,),
            in_specs=[pl.BlockSpec((1, T), index_map=lambda i: (i, 0))],
            out_specs=[pl.BlockSpec((1, T, LANES), index_map=lambda i: (i, 0, 0))],
            core_axis_name=("core", "subcore"),
            dimension_semantics=(pltpu.PARALLEL,),
        )(i_hbm, o_hbm)

    return gather_kernel(wp, ids)[:, :, :D]


# speedup vs baseline: 1.1483x; 1.0036x over previous
import jax
import jax.numpy as jnp
from jax.experimental import pallas as pl
from jax.experimental.pallas import tpu as pltpu
from jax.experimental.pallas import tpu_sc as plsc

LANES = 128


def kernel(input_ids, weight):
    B, T = input_ids.shape
    D = weight.shape[1]
    ids = input_ids.astype(jnp.int32)
    wp = jnp.pad(weight, ((0, 0), (0, LANES - D)))

    mesh = plsc.VectorSubcoreMesh(core_axis_name="core", subcore_axis_name="subcore")

    @pl.kernel(
        out_type=jax.ShapeDtypeStruct((B, T, LANES), weight.dtype),
        mesh=mesh,
    )
    def gather_kernel(w_hbm, i_hbm, o_hbm):
        def body(i0_vmem, i1_vmem, o0_vmem, o1_vmem):
            pltpu.sync_copy(w_hbm.at[i0_vmem.at[0]], o0_vmem.at[0])
            pltpu.sync_copy(w_hbm.at[i1_vmem.at[0]], o1_vmem.at[0])

        pltpu.emit_pipeline(
            body,
            grid=(B // 2,),
            in_specs=[
                pl.BlockSpec((1, T), index_map=lambda i: (2 * i, 0)),
                pl.BlockSpec((1, T), index_map=lambda i: (2 * i + 1, 0)),
            ],
            out_specs=[
                pl.BlockSpec((1, T, LANES), index_map=lambda i: (2 * i, 0, 0)),
                pl.BlockSpec((1, T, LANES), index_map=lambda i: (2 * i + 1, 0, 0)),
            ],
            core_axis_name=("core", "subcore"),
            dimension_semantics=(pltpu.PARALLEL,),
        )(i_hbm, i_hbm, o_hbm, o_hbm)

    return gather_kernel(wp, ids)[:, :, :D]
